# baseline (speedup 1.0000x reference)
import functools
import math

import jax
import jax.numpy as jnp
import numpy as np
from jax.experimental import pallas as pl
from jax.experimental.pallas import tpu as pltpu


def _dft_constants(L: int, LFP: int):
    Lf = L // 2 + 1
    f = np.arange(LFP, dtype=np.int64)[:, None]
    t = np.arange(L, dtype=np.int64)[None, :]
    ang = 2.0 * np.pi * ((f * t) % L).astype(np.float64) / L
    out = []
    for m in (np.cos(ang), np.sin(ang)):
        m[Lf:, :] = 0.0
        m32 = m.astype(np.float32)
        hi = m32.astype(jnp.bfloat16)
        lo = (m32 - hi.astype(np.float32)).astype(jnp.bfloat16)
        out.append((jnp.asarray(hi), jnp.asarray(lo)))
    return out[0], out[1]


def _dot3(ah, al, bh, bl, dn):
    kw = dict(dimension_numbers=dn, preferred_element_type=jnp.float32)
    return (jax.lax.dot_general(ah, bh, **kw)
            + jax.lax.dot_general(ah, bl, **kw)
            + jax.lax.dot_general(al, bh, **kw))


def _split_bf16(x):
    hi = x.astype(jnp.bfloat16)
    lo = (x - hi.astype(jnp.float32)).astype(jnp.bfloat16)
    return hi, lo


_FB = 256


def _corr_kernel(L, Lf, q_ref, k_ref, cth_ref, ctl_ref, sth_ref, stl_ref,
                 corr_ref):
    dh = q_ref.shape[-1]
    lfp = cth_ref.shape[0]
    x = jnp.concatenate(
        [q_ref[0, 0], k_ref[0, 0], q_ref[0, 1], k_ref[0, 1]],
        axis=1)
    xh, xl = _split_bf16(x)
    dn = (((1,), (0,)), ((), ()))
    dn2 = (((0,), (0,)), ((), ()))
    nfb = lfp // _FB

    res_re_l = []
    res_im_l = []
    for i in range(nfb):
        f0 = i * _FB
        cth = cth_ref[f0:f0 + _FB, :]
        ctl = ctl_ref[f0:f0 + _FB, :]
        sth = sth_ref[f0:f0 + _FB, :]
        stl = stl_ref[f0:f0 + _FB, :]
        a = _dot3(cth, ctl, xh, xl, dn)
        b = _dot3(sth, stl, xh, xl, dn)
        fidx = f0 + jax.lax.broadcasted_iota(jnp.int32, (_FB, 1), 0)
        w = jnp.where((fidx == 0) | (fidx == L // 2), 1.0, 2.0) / (L * dh)
        res_re = []
        res_im = []
        for h in range(2):
            aq, ak = a[:, 2 * h * dh:(2 * h + 1) * dh] * w, \
                a[:, (2 * h + 1) * dh:(2 * h + 2) * dh]
            bq, bk = b[:, 2 * h * dh:(2 * h + 1) * dh] * w, \
                b[:, (2 * h + 1) * dh:(2 * h + 2) * dh]
            res_re.append(jnp.sum(aq * ak + bq * bk, axis=1, keepdims=True))
            res_im.append(jnp.sum(aq * bk - bq * ak, axis=1, keepdims=True))
        res_re_l.append(jnp.concatenate(res_re, axis=1))
        res_im_l.append(jnp.concatenate(res_im, axis=1))

    res_re_all = jnp.concatenate(res_re_l, axis=0)
    res_im_all = jnp.concatenate(res_im_l, axis=0)
    reh, rel = _split_bf16(res_re_all)
    imh, iml = _split_bf16(res_im_all)

    acc = jnp.zeros((2, L), jnp.float32)
    for i in range(nfb):
        f0 = i * _FB
        sl = (slice(f0, f0 + _FB), slice(None))
        c_re = _dot3(reh[sl], rel[sl], cth_ref[sl], ctl_ref[sl], dn2)
        c_im = _dot3(imh[sl], iml[sl], sth_ref[sl], stl_ref[sl], dn2)
        acc = acc + c_re - c_im
    corr_ref[0, :, 0] = acc


def _agg_kernel(K, corr_ref, v_ref, out_ref, v2_ref, g_ref, sems):
    L = v_ref.shape[2]
    cp0 = pltpu.make_async_copy(v_ref.at[0, 0], v2_ref.at[pl.ds(0, L), :],
                                sems.at[K])
    cp1 = pltpu.make_async_copy(v_ref.at[0, 0], v2_ref.at[pl.ds(L, L), :],
                                sems.at[K + 1])
    cp0.start()
    cp1.start()

    r = corr_ref[0, 0]
    iota = jax.lax.broadcasted_iota(jnp.int32, r.shape, 1)
    neg = jnp.float32(-jnp.inf)
    vals = []
    idxs = []
    for _ in range(K):
        m = jnp.max(r)
        i = jnp.min(jnp.where(r == m, iota, L))
        vals.append(m)
        idxs.append(i)
        r = jnp.where(iota == i, neg, r)
    m0 = functools.reduce(jnp.maximum, vals)
    es = [jnp.exp(w - m0) for w in vals]
    s = functools.reduce(lambda x, y: x + y, es)

    cp0.wait()
    cp1.wait()
    cps = []
    for j in range(K):
        cp = pltpu.make_async_copy(v2_ref.at[pl.ds(L - idxs[j], L), :],
                                   g_ref.at[j], sems.at[j])
        cp.start()
        cps.append(cp)
    cps[0].wait()
    acc = (es[0] / s) * g_ref[0]
    for j in range(1, K):
        cps[j].wait()
        acc = acc + (es[j] / s) * g_ref[j]
    out_ref[0, 0] = acc


def kernel(q, k, v):
    B, H, L, Dh = q.shape
    BH = B * H
    Lf = L // 2 + 1
    LFP = ((Lf + _FB - 1) // _FB) * _FB
    K = max(1, int(math.log(L + 1)))

    (cth, ctl), (sth, stl) = _dft_constants(L, LFP)
    HP = H // 2

    corr = pl.pallas_call(
        functools.partial(_corr_kernel, L, Lf),
        grid=(BH // 2,),
        in_specs=[
            pl.BlockSpec((1, 2, L, Dh), lambda i: (i // HP, i % HP, 0, 0)),
            pl.BlockSpec((1, 2, L, Dh), lambda i: (i // HP, i % HP, 0, 0)),
            pl.BlockSpec((LFP, L), lambda i: (0, 0)),
            pl.BlockSpec((LFP, L), lambda i: (0, 0)),
            pl.BlockSpec((LFP, L), lambda i: (0, 0)),
            pl.BlockSpec((LFP, L), lambda i: (0, 0)),
        ],
        out_specs=pl.BlockSpec((1, 2, 1, L), lambda i: (i // HP, i % HP, 0, 0)),
        out_shape=jax.ShapeDtypeStruct((B, H, 1, L), jnp.float32),
        compiler_params=pltpu.CompilerParams(
            dimension_semantics=("arbitrary",)),
    )(q, k, cth, ctl, sth, stl)

    out = pl.pallas_call(
        functools.partial(_agg_kernel, K),
        grid=(BH,),
        in_specs=[
            pl.BlockSpec((1, 1, 1, L), lambda i: (i // H, i % H, 0, 0)),
            pl.BlockSpec((1, 1, L, Dh), lambda i: (i // H, i % H, 0, 0)),
        ],
        out_specs=pl.BlockSpec((1, 1, L, Dh), lambda i: (i // H, i % H, 0, 0)),
        out_shape=jax.ShapeDtypeStruct((B, H, L, Dh), jnp.float32),
        scratch_shapes=[
            pltpu.VMEM((2 * L, Dh), jnp.float32),
            pltpu.VMEM((K, L, Dh), jnp.float32),
            pltpu.SemaphoreType.DMA((K + 2,)),
        ],
        compiler_params=pltpu.CompilerParams(
            dimension_semantics=("arbitrary",)),
    )(corr, v)

    return out


# speedup vs baseline: 1.1467x; 1.1467x over previous
import functools
import math

import jax
import jax.numpy as jnp
import numpy as np
from jax.experimental import pallas as pl
from jax.experimental.pallas import tpu as pltpu


def _dft_constants(L2: int, LFP: int):
    Lf = L2 // 2 + 1
    f = np.arange(LFP, dtype=np.int64)[:, None]
    t = np.arange(L2, dtype=np.int64)[None, :]
    ang = 2.0 * np.pi * ((f * t) % L2).astype(np.float64) / L2
    out = []
    for m in (np.cos(ang), np.sin(ang)):
        m[Lf:, :] = 0.0
        m32 = m.astype(np.float32)
        hi = m32.astype(jnp.bfloat16)
        lo = (m32 - hi.astype(np.float32)).astype(jnp.bfloat16)
        out.append((jnp.asarray(hi), jnp.asarray(lo)))
    return out[0], out[1]


def _twiddle_constants(L2: int, LFP: int):
    f = np.arange(LFP, dtype=np.float64)[:, None]
    ang = 2.0 * np.pi * f / L2
    return (jnp.asarray(np.cos(ang), dtype=jnp.float32),
            jnp.asarray(np.sin(ang), dtype=jnp.float32))


def _dot3(ah, al, bh, bl, dn):
    kw = dict(dimension_numbers=dn, preferred_element_type=jnp.float32)
    return (jax.lax.dot_general(ah, bh, **kw)
            + jax.lax.dot_general(ah, bl, **kw)
            + jax.lax.dot_general(al, bh, **kw))


def _split_bf16(x):
    hi = x.astype(jnp.bfloat16)
    lo = (x - hi.astype(jnp.float32)).astype(jnp.bfloat16)
    return hi, lo


_FB = 128


def _corr_kernel(L, q_ref, k_ref, cth_ref, ctl_ref, sth_ref, stl_ref,
                 cu_ref, su_ref, corr_ref):
    dh2 = q_ref.shape[-1]
    dh = dh2 // 2
    lfp = cth_ref.shape[0]
    L2 = cth_ref.shape[1]
    x = jnp.concatenate(
        [q_ref[0, 0], k_ref[0, 0], q_ref[0, 1], k_ref[0, 1]],
        axis=1)
    xh, xl = _split_bf16(x)
    dn = (((1,), (0,)), ((), ()))
    dn2 = (((0,), (0,)), ((), ()))
    nfb = lfp // _FB

    res = {p: ([], []) for p in ('ee', 'oe', 'eo', 'oo')}
    for i in range(nfb):
        f0 = i * _FB
        cth = cth_ref[f0:f0 + _FB, :]
        ctl = ctl_ref[f0:f0 + _FB, :]
        sth = sth_ref[f0:f0 + _FB, :]
        stl = stl_ref[f0:f0 + _FB, :]
        a = _dot3(cth, ctl, xh, xl, dn)
        b = _dot3(sth, stl, xh, xl, dn)
        for p, (qs, ks) in (('ee', (0, 2)), ('oe', (1, 2)),
                            ('eo', (0, 3)), ('oo', (1, 3))):
            re_h = []
            im_h = []
            for h in range(2):
                base = 2 * h * dh2
                aq = a[:, base + qs * dh:base + (qs + 1) * dh]
                bq = b[:, base + qs * dh:base + (qs + 1) * dh]
                ak = a[:, base + ks * dh:base + (ks + 1) * dh]
                bk = b[:, base + ks * dh:base + (ks + 1) * dh]
                re_h.append(jnp.sum(aq * ak + bq * bk, axis=1, keepdims=True))
                im_h.append(jnp.sum(aq * bk - bq * ak, axis=1, keepdims=True))
            res[p][0].append(jnp.concatenate(re_h, axis=1))
            res[p][1].append(jnp.concatenate(im_h, axis=1))

    P = {p: (jnp.concatenate(res[p][0], axis=0),
             jnp.concatenate(res[p][1], axis=0)) for p in res}

    fidx = jax.lax.broadcasted_iota(jnp.int32, (lfp, 1), 0)
    s = jnp.where((fidx == 0) | (fidx == L2 // 2), 1.0, 2.0) * (
        2.0 / (L * dh))
    cu = cu_ref[...]
    su = su_ref[...]
    g0re = s * (P['ee'][0] + P['oo'][0])
    g0im = s * (P['ee'][1] + P['oo'][1])
    g1re = s * (P['oe'][0] + cu * P['eo'][0] - su * P['eo'][1])
    g1im = s * (P['oe'][1] + su * P['eo'][0] + cu * P['eo'][1])

    g0rh, g0rl = _split_bf16(g0re)
    g0ih, g0il = _split_bf16(g0im)
    g1rh, g1rl = _split_bf16(g1re)
    g1ih, g1il = _split_bf16(g1im)

    acc_e = jnp.zeros((2, L2), jnp.float32)
    acc_o = jnp.zeros((2, L2), jnp.float32)
    for i in range(nfb):
        f0 = i * _FB
        sl = (slice(f0, f0 + _FB), slice(None))
        acc_e = (acc_e
                 + _dot3(g0rh[sl], g0rl[sl], cth_ref[sl], ctl_ref[sl], dn2)
                 - _dot3(g0ih[sl], g0il[sl], sth_ref[sl], stl_ref[sl], dn2))
        acc_o = (acc_o
                 + _dot3(g1rh[sl], g1rl[sl], cth_ref[sl], ctl_ref[sl], dn2)
                 - _dot3(g1ih[sl], g1il[sl], sth_ref[sl], stl_ref[sl], dn2))
    corr_ref[0, :, 0] = jnp.concatenate([acc_e, acc_o], axis=1)


def _agg_kernel(K, corr_ref, v_ref, out_ref, v2_ref, g_ref, sems):
    L = v_ref.shape[2]
    L2 = L // 2
    cp0 = pltpu.make_async_copy(v_ref.at[0, 0], v2_ref.at[pl.ds(0, L), :],
                                sems.at[K])
    cp1 = pltpu.make_async_copy(v_ref.at[0, 0], v2_ref.at[pl.ds(L, L), :],
                                sems.at[K + 1])
    cp0.start()
    cp1.start()

    r = corr_ref[0, 0]
    iota = jax.lax.broadcasted_iota(jnp.int32, r.shape, 1)
    mapped = jnp.where(iota < L2, 2 * iota, 2 * iota - (L - 1))
    neg = jnp.float32(-jnp.inf)
    vals = []
    idxs = []
    for _ in range(K):
        m = jnp.max(r)
        d = jnp.min(jnp.where(r == m, mapped, L))
        vals.append(m)
        idxs.append(d)
        r = jnp.where(mapped == d, neg, r)
    m0 = functools.reduce(jnp.maximum, vals)
    es = [jnp.exp(w - m0) for w in vals]
    s = functools.reduce(lambda x, y: x + y, es)

    cp0.wait()
    cp1.wait()
    cps = []
    for j in range(K):
        cp = pltpu.make_async_copy(v2_ref.at[pl.ds(L - idxs[j], L), :],
                                   g_ref.at[j], sems.at[j])
        cp.start()
        cps.append(cp)
    cps[0].wait()
    acc = (es[0] / s) * g_ref[0]
    for j in range(1, K):
        cps[j].wait()
        acc = acc + (es[j] / s) * g_ref[j]
    out_ref[0, 0] = acc


def kernel(q, k, v):
    B, H, L, Dh = q.shape
    BH = B * H
    L2 = L // 2
    Lf2 = L2 // 2 + 1
    LFP = ((Lf2 + _FB - 1) // _FB) * _FB
    K = max(1, int(math.log(L + 1)))

    (cth, ctl), (sth, stl) = _dft_constants(L2, LFP)
    cu, su = _twiddle_constants(L2, LFP)
    q4 = q.reshape(B, H, L2, 2 * Dh)
    k4 = k.reshape(B, H, L2, 2 * Dh)
    HP = H // 2

    corr = pl.pallas_call(
        functools.partial(_corr_kernel, L),
        grid=(BH // 2,),
        in_specs=[
            pl.BlockSpec((1, 2, L2, 2 * Dh), lambda i: (i // HP, i % HP, 0, 0)),
            pl.BlockSpec((1, 2, L2, 2 * Dh), lambda i: (i // HP, i % HP, 0, 0)),
            pl.BlockSpec((LFP, L2), lambda i: (0, 0)),
            pl.BlockSpec((LFP, L2), lambda i: (0, 0)),
            pl.BlockSpec((LFP, L2), lambda i: (0, 0)),
            pl.BlockSpec((LFP, L2), lambda i: (0, 0)),
            pl.BlockSpec((LFP, 1), lambda i: (0, 0)),
            pl.BlockSpec((LFP, 1), lambda i: (0, 0)),
        ],
        out_specs=pl.BlockSpec((1, 2, 1, L), lambda i: (i // HP, i % HP, 0, 0)),
        out_shape=jax.ShapeDtypeStruct((B, H, 1, L), jnp.float32),
        compiler_params=pltpu.CompilerParams(
            dimension_semantics=("arbitrary",)),
    )(q4, k4, cth, ctl, sth, stl, cu, su)

    out = pl.pallas_call(
        functools.partial(_agg_kernel, K),
        grid=(BH,),
        in_specs=[
            pl.BlockSpec((1, 1, 1, L), lambda i: (i // H, i % H, 0, 0)),
            pl.BlockSpec((1, 1, L, Dh), lambda i: (i // H, i % H, 0, 0)),
        ],
        out_specs=pl.BlockSpec((1, 1, L, Dh), lambda i: (i // H, i % H, 0, 0)),
        out_shape=jax.ShapeDtypeStruct((B, H, L, Dh), jnp.float32),
        scratch_shapes=[
            pltpu.VMEM((2 * L, Dh), jnp.float32),
            pltpu.VMEM((K, L, Dh), jnp.float32),
            pltpu.SemaphoreType.DMA((K + 2,)),
        ],
        compiler_params=pltpu.CompilerParams(
            dimension_semantics=("arbitrary",)),
    )(corr, v)

    return out


# speedup vs baseline: 1.1496x; 1.0025x over previous
import functools
import math

import jax
import jax.numpy as jnp
import numpy as np
from jax.experimental import pallas as pl
from jax.experimental.pallas import tpu as pltpu


def _dft_constants(L2: int, LFP: int):
    Lf = L2 // 2 + 1
    f = np.arange(LFP, dtype=np.int64)[:, None]
    t = np.arange(L2, dtype=np.int64)[None, :]
    ang = 2.0 * np.pi * ((f * t) % L2).astype(np.float64) / L2
    out = []
    for m in (np.cos(ang), np.sin(ang)):
        m[Lf:, :] = 0.0
        m32 = m.astype(np.float32)
        hi = m32.astype(jnp.bfloat16)
        lo = (m32 - hi.astype(np.float32)).astype(jnp.bfloat16)
        out.append((jnp.asarray(hi), jnp.asarray(lo)))
    return out[0], out[1]


def _twiddle_constants(L2: int, LFP: int):
    f = np.arange(LFP, dtype=np.float64)[:, None]
    ang = 2.0 * np.pi * f / L2
    return (jnp.asarray(np.cos(ang), dtype=jnp.float32),
            jnp.asarray(np.sin(ang), dtype=jnp.float32))


def _dot3(ah, al, bh, bl, dn):
    kw = dict(dimension_numbers=dn, preferred_element_type=jnp.float32)
    return (jax.lax.dot_general(ah, bh, **kw)
            + jax.lax.dot_general(ah, bl, **kw)
            + jax.lax.dot_general(al, bh, **kw))


def _split_bf16(x):
    hi = x.astype(jnp.bfloat16)
    lo = (x - hi.astype(jnp.float32)).astype(jnp.bfloat16)
    return hi, lo


_FB = 128


def _fused_kernel(L, K, q_ref, k_ref, v_ref, cth_ref, ctl_ref, sth_ref,
                  stl_ref, cu_ref, su_ref, out_ref, v2_ref, g_ref, sems):
    dh2 = q_ref.shape[-1]
    dh = dh2 // 2
    lfp = cth_ref.shape[0]
    L2 = cth_ref.shape[1]
    fills = []
    for h in range(2):
        cpa = pltpu.make_async_copy(v_ref.at[0, h],
                                    v2_ref.at[h, pl.ds(0, L), :],
                                    sems.at[h, K])
        cpb = pltpu.make_async_copy(v_ref.at[0, h],
                                    v2_ref.at[h, pl.ds(L, L), :],
                                    sems.at[h, K + 1])
        cpa.start()
        cpb.start()
        fills += [cpa, cpb]
    x = jnp.concatenate(
        [q_ref[0, 0], k_ref[0, 0], q_ref[0, 1], k_ref[0, 1]],
        axis=1)
    xh, xl = _split_bf16(x)
    dn = (((1,), (0,)), ((), ()))
    dn2 = (((0,), (0,)), ((), ()))
    nfb = lfp // _FB

    res = {p: ([], []) for p in ('ee', 'oe', 'eo', 'oo')}
    for i in range(nfb):
        f0 = i * _FB
        cth = cth_ref[f0:f0 + _FB, :]
        ctl = ctl_ref[f0:f0 + _FB, :]
        sth = sth_ref[f0:f0 + _FB, :]
        stl = stl_ref[f0:f0 + _FB, :]
        a = _dot3(cth, ctl, xh, xl, dn)
        b = _dot3(sth, stl, xh, xl, dn)
        for p, (qs, ks) in (('ee', (0, 2)), ('oe', (1, 2)),
                            ('eo', (0, 3)), ('oo', (1, 3))):
            re_h = []
            im_h = []
            for h in range(2):
                base = 2 * h * dh2
                aq = a[:, base + qs * dh:base + (qs + 1) * dh]
                bq = b[:, base + qs * dh:base + (qs + 1) * dh]
                ak = a[:, base + ks * dh:base + (ks + 1) * dh]
                bk = b[:, base + ks * dh:base + (ks + 1) * dh]
                re_h.append(jnp.sum(aq * ak + bq * bk, axis=1, keepdims=True))
                im_h.append(jnp.sum(aq * bk - bq * ak, axis=1, keepdims=True))
            res[p][0].append(jnp.concatenate(re_h, axis=1))
            res[p][1].append(jnp.concatenate(im_h, axis=1))

    P = {p: (jnp.concatenate(res[p][0], axis=0),
             jnp.concatenate(res[p][1], axis=0)) for p in res}

    fidx = jax.lax.broadcasted_iota(jnp.int32, (lfp, 1), 0)
    s = jnp.where((fidx == 0) | (fidx == L2 // 2), 1.0, 2.0) * (
        2.0 / (L * dh))
    cu = cu_ref[...]
    su = su_ref[...]
    g0re = s * (P['ee'][0] + P['oo'][0])
    g0im = s * (P['ee'][1] + P['oo'][1])
    g1re = s * (P['oe'][0] + cu * P['eo'][0] - su * P['eo'][1])
    g1im = s * (P['oe'][1] + su * P['eo'][0] + cu * P['eo'][1])

    g0rh, g0rl = _split_bf16(g0re)
    g0ih, g0il = _split_bf16(g0im)
    g1rh, g1rl = _split_bf16(g1re)
    g1ih, g1il = _split_bf16(g1im)

    acc_e = jnp.zeros((2, L2), jnp.float32)
    acc_o = jnp.zeros((2, L2), jnp.float32)
    for i in range(nfb):
        f0 = i * _FB
        sl = (slice(f0, f0 + _FB), slice(None))
        acc_e = (acc_e
                 + _dot3(g0rh[sl], g0rl[sl], cth_ref[sl], ctl_ref[sl], dn2)
                 - _dot3(g0ih[sl], g0il[sl], sth_ref[sl], stl_ref[sl], dn2))
        acc_o = (acc_o
                 + _dot3(g1rh[sl], g1rl[sl], cth_ref[sl], ctl_ref[sl], dn2)
                 - _dot3(g1ih[sl], g1il[sl], sth_ref[sl], stl_ref[sl], dn2))
    iota = jax.lax.broadcasted_iota(jnp.int32, (1, L), 1)
    mapped = jnp.where(iota < L2, 2 * iota, 2 * iota - (L - 1))
    neg = jnp.float32(-jnp.inf)

    def topk(r):
        vals = []
        idxs = []
        for _ in range(K):
            m = jnp.max(r)
            d = jnp.min(jnp.where(r == m, mapped, L))
            vals.append(m)
            idxs.append(d)
            r = jnp.where(mapped == d, neg, r)
        m0 = functools.reduce(jnp.maximum, vals)
        es = [jnp.exp(w - m0) for w in vals]
        ssum = functools.reduce(lambda x, y: x + y, es)
        return [e / ssum for e in es], idxs

    def start_gathers(h, idxs):
        for cp in fills[2 * h:2 * h + 2]:
            cp.wait()
        cps = []
        for j in range(K):
            cp = pltpu.make_async_copy(
                v2_ref.at[h, pl.ds(L - idxs[j], L), :],
                g_ref.at[h, j], sems.at[h, j])
            cp.start()
            cps.append(cp)
        return cps

    def accumulate(h, attn, cps):
        cps[0].wait()
        acc = attn[0] * g_ref[h, 0]
        for j in range(1, K):
            cps[j].wait()
            acc = acc + attn[j] * g_ref[h, j]
        out_ref[0, h] = acc

    r0 = jnp.concatenate([acc_e[0:1], acc_o[0:1]], axis=1)
    r1 = jnp.concatenate([acc_e[1:2], acc_o[1:2]], axis=1)
    attn0, idxs0 = topk(r0)
    cps0 = start_gathers(0, idxs0)
    attn1, idxs1 = topk(r1)
    cps1 = start_gathers(1, idxs1)
    accumulate(0, attn0, cps0)
    accumulate(1, attn1, cps1)


def kernel(q, k, v):
    B, H, L, Dh = q.shape
    BH = B * H
    L2 = L // 2
    Lf2 = L2 // 2 + 1
    LFP = ((Lf2 + _FB - 1) // _FB) * _FB
    K = max(1, int(math.log(L + 1)))

    (cth, ctl), (sth, stl) = _dft_constants(L2, LFP)
    cu, su = _twiddle_constants(L2, LFP)
    q4 = q.reshape(B, H, L2, 2 * Dh)
    k4 = k.reshape(B, H, L2, 2 * Dh)
    HP = H // 2

    out = pl.pallas_call(
        functools.partial(_fused_kernel, L, K),
        grid=(BH // 2,),
        in_specs=[
            pl.BlockSpec((1, 2, L2, 2 * Dh), lambda i: (i // HP, i % HP, 0, 0)),
            pl.BlockSpec((1, 2, L2, 2 * Dh), lambda i: (i // HP, i % HP, 0, 0)),
            pl.BlockSpec((1, 2, L, Dh), lambda i: (i // HP, i % HP, 0, 0)),
            pl.BlockSpec((LFP, L2), lambda i: (0, 0)),
            pl.BlockSpec((LFP, L2), lambda i: (0, 0)),
            pl.BlockSpec((LFP, L2), lambda i: (0, 0)),
            pl.BlockSpec((LFP, L2), lambda i: (0, 0)),
            pl.BlockSpec((LFP, 1), lambda i: (0, 0)),
            pl.BlockSpec((LFP, 1), lambda i: (0, 0)),
        ],
        out_specs=pl.BlockSpec((1, 2, L, Dh), lambda i: (i // HP, i % HP, 0, 0)),
        out_shape=jax.ShapeDtypeStruct((B, H, L, Dh), jnp.float32),
        scratch_shapes=[
            pltpu.VMEM((2, 2 * L, Dh), jnp.float32),
            pltpu.VMEM((2, K, L, Dh), jnp.float32),
            pltpu.SemaphoreType.DMA((2, K + 2)),
        ],
        compiler_params=pltpu.CompilerParams(
            dimension_semantics=("arbitrary",)),
    )(q4, k4, v, cth, ctl, sth, stl, cu, su)

    return out
